# Initial kernel scaffold; baseline (speedup 1.0000x reference)
import functools

import jax
import jax.numpy as jnp
from jax import lax
from jax.experimental import pallas as pl
from jax.experimental.pallas import tpu as pltpu
from jax.experimental.pallas import tpu_sc as plsc

N = 10000
E = 320000
NW = 32
CHUNK = 128
CH = 80
EPW = CH * CHUNK
E_PAD = NW * EPW
NPAD = 10016
STRIPE = NPAD // 16

_mesh = plsc.VectorSubcoreMesh(core_axis_name="c", subcore_axis_name="s")


@functools.partial(
    pl.kernel,
    out_type=jax.ShapeDtypeStruct((2, NPAD, 8), jnp.float32),
    mesh=_mesh,
    scratch_types=[
        pltpu.VMEM((CH, CHUNK), jnp.int32),
        pltpu.VMEM((CHUNK, 8), jnp.float32),
        pltpu.VMEM_SHARED((NPAD, 8), jnp.float32),
    ],
)
def _deg_kernel(dstq_hbm, zeros_hbm, ones_hbm, out_hbm, didx, ones_v, shared):
    c = lax.axis_index("c")
    s = lax.axis_index("s")
    wid = s * 2 + c
    r0 = s * STRIPE
    pltpu.sync_copy(zeros_hbm.at[pl.ds(r0, STRIPE)], shared.at[pl.ds(r0, STRIPE)])
    pltpu.sync_copy(dstq_hbm.at[pl.ds(wid * CH, CH)], didx)
    pltpu.sync_copy(ones_hbm, ones_v)
    plsc.subcore_barrier()

    def body(j, carry):
        pltpu.sync_copy(ones_v, shared.at[didx.at[j]], add=True)
        return carry

    lax.fori_loop(0, CH, body, 0)
    plsc.subcore_barrier()
    pltpu.sync_copy(shared.at[pl.ds(r0, STRIPE)], out_hbm.at[c, pl.ds(r0, STRIPE)])


def _make_edge_kernel(F):
    @functools.partial(
        pl.kernel,
        out_type=jax.ShapeDtypeStruct((2, NPAD, F), jnp.float32),
        mesh=_mesh,
        scratch_types=[
            pltpu.VMEM((CH, CHUNK), jnp.int32),
            pltpu.VMEM((CH, CHUNK), jnp.int32),
            pltpu.VMEM((CHUNK, F), jnp.float32),
            pltpu.VMEM((CHUNK, F), jnp.float32),
            pltpu.SemaphoreType.DMA,
            pltpu.SemaphoreType.DMA,
            pltpu.VMEM_SHARED((NPAD, F), jnp.float32),
        ],
    )
    def edge_kernel(g_hbm, srcq_hbm, dstq_hbm, zeros_hbm, out_hbm,
                    sidx, didx, rows0, rows1, sem0, sem1, shared):
        c = lax.axis_index("c")
        s = lax.axis_index("s")
        wid = s * 2 + c
        r0 = s * STRIPE
        pltpu.sync_copy(zeros_hbm.at[pl.ds(r0, STRIPE)], shared.at[pl.ds(r0, STRIPE)])
        pltpu.sync_copy(srcq_hbm.at[pl.ds(wid * CH, CH)], sidx)
        pltpu.sync_copy(dstq_hbm.at[pl.ds(wid * CH, CH)], didx)
        plsc.subcore_barrier()

        rows = (rows0, rows1)
        sems = (sem0, sem1)
        pltpu.async_copy(g_hbm.at[sidx.at[0]], rows0, sem0)
        pltpu.async_copy(g_hbm.at[sidx.at[1]], rows1, sem1)

        def body(i, carry):
            for b in range(2):
                j = i * 2 + b
                pltpu.make_async_copy(g_hbm.at[sidx.at[j]], rows[b], sems[b]).wait()
                pltpu.sync_copy(rows[b], shared.at[didx.at[j]], add=True)
                pltpu.async_copy(g_hbm.at[sidx.at[j + 2]], rows[b], sems[b])
            return carry

        lax.fori_loop(0, CH // 2 - 1, body, 0)
        for b in range(2):
            j = CH - 2 + b
            pltpu.make_async_copy(g_hbm.at[sidx.at[j]], rows[b], sems[b]).wait()
            pltpu.sync_copy(rows[b], shared.at[didx.at[j]], add=True)
        plsc.subcore_barrier()
        pltpu.sync_copy(shared.at[pl.ds(r0, STRIPE)], out_hbm.at[c, pl.ds(r0, STRIPE)])

    return edge_kernel


_edge32 = _make_edge_kernel(32)
_edge64 = _make_edge_kernel(64)


def _tc_a_body(x_ref, w1_ref, degp_ref, g1_ref, dinv_ref):
    degp = degp_ref[:]
    deg = degp[0, :N, 0:1] + degp[1, :N, 0:1] + 1.0
    dinv = lax.rsqrt(deg)
    h1 = jnp.dot(x_ref[:], w1_ref[:], preferred_element_type=jnp.float32)
    g1_ref[:] = h1 * dinv
    dinv_ref[:] = dinv


def _tc_c_body(s1p_ref, g1_ref, dinv_ref, b1_ref, w2_ref, g2_ref):
    s1p = s1p_ref[:]
    dinv = dinv_ref[:]
    g1 = g1_ref[:]
    h1 = dinv * (s1p[0, :N] + s1p[1, :N] + g1) + b1_ref[:]
    h2 = jnp.dot(h1, w2_ref[:], preferred_element_type=jnp.float32)
    g2_ref[:] = h2 * dinv


def _tc_e_body(s2p_ref, g2_ref, dinv_ref, b2_ref, batch_ref,
               wf1_ref, bf1_ref, wf2_ref, bf2_ref, out_ref):
    s2p = s2p_ref[:]
    dinv = dinv_ref[:]
    h2 = dinv * (s2p[0, :N] + s2p[1, :N] + g2_ref[:]) + b2_ref[:]
    ids = lax.broadcasted_iota(jnp.int32, (64, N), 0)
    oh = (ids == batch_ref[:]).astype(jnp.float32)
    sums = jnp.dot(oh, h2, preferred_element_type=jnp.float32)
    cnts = jnp.sum(oh, axis=1, keepdims=True)
    pooled = sums / jnp.maximum(cnts, 1.0)
    z = jnp.maximum(jnp.dot(pooled, wf1_ref[:],
                            preferred_element_type=jnp.float32) + bf1_ref[:], 0.0)
    y = jnp.dot(z, wf2_ref[:], preferred_element_type=jnp.float32) + bf2_ref[:]
    out_ref[:] = jnp.maximum(y, 0.0) + jnp.log1p(jnp.exp(-jnp.abs(y)))


def kernel(x, edge_index, batch, W1, b1, W2, b2, Wf1, bf1, Wf2, bf2):
    src = edge_index[0]
    dst = edge_index[1]
    pad = E_PAD - E
    pr = jnp.arange(pad, dtype=jnp.int32)
    srcq = jnp.concatenate([src, pr % N]).reshape(NW * CH, CHUNK)
    dstq = jnp.concatenate([dst, N + (pr % 16)]).reshape(NW * CH, CHUNK)

    zeros8 = jnp.zeros((NPAD, 8), jnp.float32)
    zeros32 = jnp.zeros((NPAD, 32), jnp.float32)
    zeros64 = jnp.zeros((NPAD, 64), jnp.float32)
    ones8 = jnp.ones((CHUNK, 8), jnp.float32)

    degp = _deg_kernel(dstq, zeros8, ones8)

    g1, dinv = pl.pallas_call(
        _tc_a_body,
        out_shape=(jax.ShapeDtypeStruct((N, 32), jnp.float32),
                   jax.ShapeDtypeStruct((N, 1), jnp.float32)),
    )(x, W1, degp)

    s1p = _edge32(g1, srcq, dstq, zeros32)

    g2 = pl.pallas_call(
        _tc_c_body,
        out_shape=jax.ShapeDtypeStruct((N, 64), jnp.float32),
    )(s1p, g1, dinv, b1.reshape(1, 32), W2)

    s2p = _edge64(g2, srcq, dstq, zeros64)

    out = pl.pallas_call(
        _tc_e_body,
        out_shape=jax.ShapeDtypeStruct((64, 10), jnp.float32),
    )(s2p, g2, dinv, b2.reshape(1, 64), batch.reshape(1, N),
      Wf1, bf1.reshape(1, 32), Wf2, bf2.reshape(1, 10))

    return out


# speedup vs baseline: 41.7750x; 41.7750x over previous
import functools

import jax
import jax.numpy as jnp
from jax import lax
from jax.experimental import pallas as pl
from jax.experimental.pallas import tpu as pltpu
from jax.experimental.pallas import tpu_sc as plsc

N = 10000
E = 320000
NW = 32
CHUNK = 128
CH = 80
EPW = CH * CHUNK
E_PAD = NW * EPW
NPAD = 10112
STRIPE = NPAD // 16

_mesh = plsc.VectorSubcoreMesh(core_axis_name="c", subcore_axis_name="s")


@functools.partial(
    pl.kernel,
    out_type=jax.ShapeDtypeStruct((2, NPAD, 16), jnp.float32),
    mesh=_mesh,
    scratch_types=[
        pltpu.VMEM((CH, CHUNK), jnp.int32),
        pltpu.VMEM((CHUNK, 16), jnp.float32),
        pltpu.VMEM_SHARED((NPAD, 16), jnp.float32),
    ],
    compiler_params=pltpu.CompilerParams(use_tc_tiling_on_sc=False),
)
def _deg_kernel(dstq_hbm, zeros_hbm, ones_hbm, out_hbm, didx, ones_v, shared):
    c = lax.axis_index("c")
    s = lax.axis_index("s")
    wid = s * 2 + c
    r0 = s * STRIPE
    pltpu.sync_copy(zeros_hbm.at[pl.ds(r0, STRIPE)], shared.at[pl.ds(r0, STRIPE)])
    pltpu.sync_copy(dstq_hbm.at[pl.ds(wid * CH, CH)], didx)
    pltpu.sync_copy(ones_hbm, ones_v)
    plsc.subcore_barrier()

    def body(j, carry):
        pltpu.sync_copy(ones_v, shared.at[didx.at[j]], add=True)
        return carry

    lax.fori_loop(0, CH, body, 0)
    plsc.subcore_barrier()
    pltpu.sync_copy(shared.at[pl.ds(r0, STRIPE)], out_hbm.at[c, pl.ds(r0, STRIPE)])


def _make_edge_kernel(F):
    @functools.partial(
        pl.kernel,
        out_type=jax.ShapeDtypeStruct((2, NPAD, F), jnp.float32),
        mesh=_mesh,
        scratch_types=[
            pltpu.VMEM((CH, CHUNK), jnp.int32),
            pltpu.VMEM((CH, CHUNK), jnp.int32),
            pltpu.VMEM((CHUNK, F), jnp.float32),
            pltpu.VMEM((CHUNK, F), jnp.float32),
            pltpu.SemaphoreType.DMA,
            pltpu.SemaphoreType.DMA,
            pltpu.VMEM_SHARED((NPAD, F), jnp.float32),
        ],
        compiler_params=pltpu.CompilerParams(use_tc_tiling_on_sc=False),
    )
    def edge_kernel(g_hbm, srcq_hbm, dstq_hbm, zeros_hbm, out_hbm,
                    sidx, didx, rows0, rows1, sem0, sem1, shared):
        c = lax.axis_index("c")
        s = lax.axis_index("s")
        wid = s * 2 + c
        r0 = s * STRIPE
        pltpu.sync_copy(zeros_hbm.at[pl.ds(r0, STRIPE)], shared.at[pl.ds(r0, STRIPE)])
        pltpu.sync_copy(srcq_hbm.at[pl.ds(wid * CH, CH)], sidx)
        pltpu.sync_copy(dstq_hbm.at[pl.ds(wid * CH, CH)], didx)
        plsc.subcore_barrier()

        rows = (rows0, rows1)
        sems = (sem0, sem1)
        pltpu.async_copy(g_hbm.at[sidx.at[0]], rows0, sem0)
        pltpu.async_copy(g_hbm.at[sidx.at[1]], rows1, sem1)

        def body(i, carry):
            for b in range(2):
                j = i * 2 + b
                pltpu.make_async_copy(g_hbm.at[sidx.at[j]], rows[b], sems[b]).wait()
                pltpu.sync_copy(rows[b], shared.at[didx.at[j]], add=True)
                pltpu.async_copy(g_hbm.at[sidx.at[j + 2]], rows[b], sems[b])
            return carry

        lax.fori_loop(0, CH // 2 - 1, body, 0)
        for b in range(2):
            j = CH - 2 + b
            pltpu.make_async_copy(g_hbm.at[sidx.at[j]], rows[b], sems[b]).wait()
            pltpu.sync_copy(rows[b], shared.at[didx.at[j]], add=True)
        plsc.subcore_barrier()
        pltpu.sync_copy(shared.at[pl.ds(r0, STRIPE)], out_hbm.at[c, pl.ds(r0, STRIPE)])

    return edge_kernel


_edge32 = _make_edge_kernel(32)
_edge64 = _make_edge_kernel(64)


def _tc_a_body(x_ref, w1_ref, degp_ref, g1_ref, dinv_ref):
    degp = degp_ref[:]
    deg = degp[0, :N, 0:1] + degp[1, :N, 0:1] + 1.0
    dinv = lax.rsqrt(deg)
    h1 = jnp.dot(x_ref[:], w1_ref[:], preferred_element_type=jnp.float32)
    g1_ref[:] = h1 * dinv
    dinv_ref[:] = dinv


def _tc_c_body(s1p_ref, g1_ref, dinv_ref, b1_ref, w2_ref, g2_ref):
    s1p = s1p_ref[:]
    dinv = dinv_ref[:]
    g1 = g1_ref[:]
    h1 = dinv * (s1p[0, :N] + s1p[1, :N] + g1) + b1_ref[:]
    h2 = jnp.dot(h1, w2_ref[:], preferred_element_type=jnp.float32)
    g2_ref[:] = h2 * dinv


def _tc_e_body(s2p_ref, g2_ref, dinv_ref, b2_ref, batch_ref,
               wf1_ref, bf1_ref, wf2_ref, bf2_ref, out_ref):
    s2p = s2p_ref[:]
    dinv = dinv_ref[:]
    h2 = dinv * (s2p[0, :N] + s2p[1, :N] + g2_ref[:]) + b2_ref[:]
    ids = lax.broadcasted_iota(jnp.int32, (64, N), 0)
    oh = (ids == batch_ref[:]).astype(jnp.float32)
    sums = jnp.dot(oh, h2, preferred_element_type=jnp.float32)
    cnts = jnp.sum(oh, axis=1, keepdims=True)
    pooled = sums / jnp.maximum(cnts, 1.0)
    z = jnp.maximum(jnp.dot(pooled, wf1_ref[:],
                            preferred_element_type=jnp.float32) + bf1_ref[:], 0.0)
    y = jnp.dot(z, wf2_ref[:], preferred_element_type=jnp.float32) + bf2_ref[:]
    out_ref[:] = jnp.maximum(y, 0.0) + jnp.log1p(jnp.exp(-jnp.abs(y)))


def kernel(x, edge_index, batch, W1, b1, W2, b2, Wf1, bf1, Wf2, bf2):
    src = edge_index[0]
    dst = edge_index[1]
    pad = E_PAD - E
    pr = jnp.arange(pad, dtype=jnp.int32)
    srcq = jnp.concatenate([src, pr % N]).reshape(NW * CH, CHUNK)
    dstq = jnp.concatenate([dst, N + (pr % (NPAD - N))]).reshape(NW * CH, CHUNK)

    zeros16 = jnp.zeros((NPAD, 16), jnp.float32)
    zeros32 = jnp.zeros((NPAD, 32), jnp.float32)
    zeros64 = jnp.zeros((NPAD, 64), jnp.float32)
    ones16 = jnp.ones((CHUNK, 16), jnp.float32)

    degp = _deg_kernel(dstq, zeros16, ones16)

    g1, dinv = pl.pallas_call(
        _tc_a_body,
        out_shape=(jax.ShapeDtypeStruct((N, 32), jnp.float32),
                   jax.ShapeDtypeStruct((N, 1), jnp.float32)),
    )(x, W1, degp)

    s1p = _edge32(g1, srcq, dstq, zeros32)

    g2 = pl.pallas_call(
        _tc_c_body,
        out_shape=jax.ShapeDtypeStruct((N, 64), jnp.float32),
    )(s1p, g1, dinv, b1.reshape(1, 32), W2)

    s2p = _edge64(g2, srcq, dstq, zeros64)

    out = pl.pallas_call(
        _tc_e_body,
        out_shape=jax.ShapeDtypeStruct((64, 10), jnp.float32),
    )(s2p, g2, dinv, b2.reshape(1, 64), batch.reshape(1, N),
      Wf1, bf1.reshape(1, 32), Wf2, bf2.reshape(1, 10))

    return out


# speedup vs baseline: 52.8543x; 1.2652x over previous
import functools

import jax
import jax.numpy as jnp
from jax import lax
from jax.experimental import pallas as pl
from jax.experimental.pallas import tpu as pltpu
from jax.experimental.pallas import tpu_sc as plsc

N = 10000
E = 320000
NW = 32
CHUNK = 128
CH = 80
E_PAD = NW * CH * CHUNK
NPAD = 10112
STRIPE = NPAD // 16
LEAD = 8

_mesh = plsc.VectorSubcoreMesh(core_axis_name="c", subcore_axis_name="s")
_sc_params = pltpu.CompilerParams(use_tc_tiling_on_sc=False)


@functools.partial(
    pl.kernel,
    out_type=jax.ShapeDtypeStruct((2, NPAD, 16), jnp.float32),
    mesh=_mesh,
    scratch_types=[
        pltpu.VMEM((CH, CHUNK), jnp.int32),
        pltpu.VMEM((CHUNK, 16), jnp.float32),
        pltpu.SemaphoreType.DMA,
        pltpu.VMEM_SHARED((NPAD, 16), jnp.float32),
    ],
    compiler_params=_sc_params,
)
def _deg_kernel(dstq_hbm, zeros_hbm, ones_hbm, out_hbm, didx, ones_v, sem, shared):
    c = lax.axis_index("c")
    s = lax.axis_index("s")
    wid = s * 2 + c
    r0 = s * STRIPE
    pltpu.sync_copy(zeros_hbm.at[pl.ds(r0, STRIPE)], shared.at[pl.ds(r0, STRIPE)])
    pltpu.sync_copy(dstq_hbm.at[pl.ds(wid * CH, CH)], didx)
    pltpu.sync_copy(ones_hbm, ones_v)
    plsc.subcore_barrier()

    for j in range(LEAD):
        pltpu.make_async_copy(ones_v, shared.at[didx.at[j]], sem).start(add=True)

    def body(j, carry):
        pltpu.make_async_copy(ones_v, shared.at[didx.at[j]], sem).wait()
        pltpu.make_async_copy(ones_v, shared.at[didx.at[j + LEAD]], sem).start(add=True)
        return carry

    lax.fori_loop(0, CH - LEAD, body, 0)
    for j in range(CH - LEAD, CH):
        pltpu.make_async_copy(ones_v, shared.at[didx.at[j]], sem).wait()
    plsc.subcore_barrier()
    pltpu.sync_copy(shared.at[pl.ds(r0, STRIPE)], out_hbm.at[c, pl.ds(r0, STRIPE)])


F = 32

@functools.partial(
    pl.kernel,
    out_type=jax.ShapeDtypeStruct((2, NPAD, F), jnp.float32),
    mesh=_mesh,
    scratch_types=[
        pltpu.VMEM((CH, CHUNK), jnp.int32),
        pltpu.VMEM((CH, CHUNK), jnp.int32),
        pltpu.VMEM((CHUNK, F), jnp.float32),
        pltpu.VMEM((CHUNK, F), jnp.float32),
        pltpu.VMEM((CHUNK, F), jnp.float32),
        pltpu.VMEM((CHUNK, F), jnp.float32),
        pltpu.SemaphoreType.DMA,
        pltpu.SemaphoreType.DMA,
        pltpu.VMEM_SHARED((NPAD, F), jnp.float32),
    ],
    compiler_params=_sc_params,
)
def _edge_kernel(g_hbm, srcq_hbm, dstq_hbm, zeros_hbm, out_hbm,
                 sidx, didx, rb0, rb1, rb2, rb3, semg, sems, shared):
    c = lax.axis_index("c")
    s = lax.axis_index("s")
    wid = s * 2 + c
    r0 = s * STRIPE
    pltpu.sync_copy(zeros_hbm.at[pl.ds(r0, STRIPE)], shared.at[pl.ds(r0, STRIPE)])
    pltpu.sync_copy(srcq_hbm.at[pl.ds(wid * CH, CH)], sidx)
    pltpu.sync_copy(dstq_hbm.at[pl.ds(wid * CH, CH)], didx)
    plsc.subcore_barrier()

    rows = (rb0, rb1, rb2, rb3)
    for b in range(4):
        pltpu.make_async_copy(g_hbm.at[sidx.at[b]], rows[b], semg).start()

    def body(i, carry):
        for b in range(4):
            m = i * 4 + b
            pltpu.make_async_copy(g_hbm.at[sidx.at[m]], rows[b], semg).wait()
            pltpu.make_async_copy(rows[b], shared.at[didx.at[m]], sems).start(add=True)
        for b in range(4):
            m = i * 4 + b
            pltpu.make_async_copy(rows[b], shared.at[didx.at[m]], sems).wait()
            pltpu.make_async_copy(g_hbm.at[sidx.at[m + 4]], rows[b], semg).start()
        return carry

    lax.fori_loop(0, CH // 4 - 1, body, 0)
    for b in range(4):
        m = CH - 4 + b
        pltpu.make_async_copy(g_hbm.at[sidx.at[m]], rows[b], semg).wait()
        pltpu.make_async_copy(rows[b], shared.at[didx.at[m]], sems).start(add=True)
    for b in range(4):
        m = CH - 4 + b
        pltpu.make_async_copy(rows[b], shared.at[didx.at[m]], sems).wait()
    plsc.subcore_barrier()
    pltpu.sync_copy(shared.at[pl.ds(r0, STRIPE)], out_hbm.at[c, pl.ds(r0, STRIPE)])


def _tc_mm1_body(x_ref, w1_ref, h1_ref):
    h1_ref[:] = jnp.dot(x_ref[:], w1_ref[:], preferred_element_type=jnp.float32)


def _tc_scale_body(h1_ref, degp_ref, g1_ref, dinv_ref):
    degp = degp_ref[:]
    deg = degp[0, :N, 0:1] + degp[1, :N, 0:1] + 1.0
    dinv = lax.rsqrt(deg)
    g1_ref[:] = h1_ref[:] * dinv
    dinv_ref[:] = dinv


def _tc_c_body(t1p_ref, g1_ref, dinv_ref, b1_ref, u2_ref):
    t1p = t1p_ref[:]
    dinv = dinv_ref[:]
    g1 = g1_ref[:]
    h1 = dinv * (t1p[0, :N] + t1p[1, :N] + g1) + b1_ref[:]
    u2_ref[:] = h1 * dinv


def _tc_e_body(t2p_ref, u2_ref, dinv_ref, w2_ref, b2_ref, batch_ref,
               wf1_ref, bf1_ref, wf2_ref, bf2_ref, out_ref):
    t2p = t2p_ref[:]
    dinv = dinv_ref[:]
    m2 = dinv * (t2p[0, :N] + t2p[1, :N] + u2_ref[:])
    h2 = jnp.dot(m2, w2_ref[:], preferred_element_type=jnp.float32) + b2_ref[:]
    ids = lax.broadcasted_iota(jnp.int32, (64, N), 0)
    oh = (ids == batch_ref[:]).astype(jnp.float32)
    sums = jnp.dot(oh, h2, preferred_element_type=jnp.float32)
    cnts = jnp.sum(oh, axis=1, keepdims=True)
    pooled = sums / jnp.maximum(cnts, 1.0)
    z = jnp.maximum(jnp.dot(pooled, wf1_ref[:],
                            preferred_element_type=jnp.float32) + bf1_ref[:], 0.0)
    y = jnp.dot(z, wf2_ref[:], preferred_element_type=jnp.float32) + bf2_ref[:]
    out_ref[:] = jnp.maximum(y, 0.0) + jnp.log1p(jnp.exp(-jnp.abs(y)))


def kernel(x, edge_index, batch, W1, b1, W2, b2, Wf1, bf1, Wf2, bf2):
    src = edge_index[0]
    dst = edge_index[1]
    pad = E_PAD - E
    pr = jnp.arange(pad, dtype=jnp.int32)
    srcq = jnp.concatenate([src, pr % N]).reshape(NW * CH, CHUNK)
    dstq = jnp.concatenate([dst, N + (pr % (NPAD - N))]).reshape(NW * CH, CHUNK)

    zeros16 = jnp.zeros((NPAD, 16), jnp.float32)
    zeros32 = jnp.zeros((NPAD, 32), jnp.float32)
    ones16 = jnp.ones((CHUNK, 16), jnp.float32)

    degp = _deg_kernel(dstq, zeros16, ones16)
    h1 = pl.pallas_call(
        _tc_mm1_body,
        out_shape=jax.ShapeDtypeStruct((N, 32), jnp.float32),
    )(x, W1)

    g1, dinv = pl.pallas_call(
        _tc_scale_body,
        out_shape=(jax.ShapeDtypeStruct((N, 32), jnp.float32),
                   jax.ShapeDtypeStruct((N, 1), jnp.float32)),
    )(h1, degp)

    t1p = _edge_kernel(g1, srcq, dstq, zeros32)

    u2 = pl.pallas_call(
        _tc_c_body,
        out_shape=jax.ShapeDtypeStruct((N, 32), jnp.float32),
    )(t1p, g1, dinv, b1.reshape(1, 32))

    t2p = _edge_kernel(u2, srcq, dstq, zeros32)

    out = pl.pallas_call(
        _tc_e_body,
        out_shape=jax.ShapeDtypeStruct((64, 10), jnp.float32),
    )(t2p, u2, dinv, W2, b2.reshape(1, 64), batch.reshape(1, N),
      Wf1, bf1.reshape(1, 32), Wf2, bf2.reshape(1, 10))

    return out


# speedup vs baseline: 54.9740x; 1.0401x over previous
import functools

import jax
import jax.numpy as jnp
from jax import lax
from jax.experimental import pallas as pl
from jax.experimental.pallas import tpu as pltpu
from jax.experimental.pallas import tpu_sc as plsc

N = 10000
E = 320000
NW = 32
CHUNK = 256
CH = 40
E_PAD = NW * CH * CHUNK
NPAD = 10112
STRIPE = NPAD // 16
LEAD = 8

_mesh = plsc.VectorSubcoreMesh(core_axis_name="c", subcore_axis_name="s")
_sc_params = pltpu.CompilerParams(use_tc_tiling_on_sc=False)


@functools.partial(
    pl.kernel,
    out_type=jax.ShapeDtypeStruct((2, NPAD, 16), jnp.float32),
    mesh=_mesh,
    scratch_types=[
        pltpu.VMEM((CH, CHUNK), jnp.int32),
        pltpu.VMEM((CHUNK, 16), jnp.float32),
        pltpu.SemaphoreType.DMA,
        pltpu.VMEM_SHARED((NPAD, 16), jnp.float32),
    ],
    compiler_params=_sc_params,
)
def _deg_kernel(dstq_hbm, zeros_hbm, ones_hbm, out_hbm, didx, ones_v, sem, shared):
    c = lax.axis_index("c")
    s = lax.axis_index("s")
    wid = s * 2 + c
    r0 = s * STRIPE
    pltpu.sync_copy(zeros_hbm.at[pl.ds(r0, STRIPE)], shared.at[pl.ds(r0, STRIPE)])
    pltpu.sync_copy(dstq_hbm.at[pl.ds(wid * CH, CH)], didx)
    pltpu.sync_copy(ones_hbm, ones_v)
    plsc.subcore_barrier()

    for j in range(LEAD):
        pltpu.make_async_copy(ones_v, shared.at[didx.at[j]], sem).start(add=True)

    def body(j, carry):
        pltpu.make_async_copy(ones_v, shared.at[didx.at[j]], sem).wait()
        pltpu.make_async_copy(ones_v, shared.at[didx.at[j + LEAD]], sem).start(add=True)
        return carry

    lax.fori_loop(0, CH - LEAD, body, 0)
    for j in range(CH - LEAD, CH):
        pltpu.make_async_copy(ones_v, shared.at[didx.at[j]], sem).wait()
    plsc.subcore_barrier()
    pltpu.sync_copy(shared.at[pl.ds(r0, STRIPE)], out_hbm.at[c, pl.ds(r0, STRIPE)])


F = 32

@functools.partial(
    pl.kernel,
    out_type=jax.ShapeDtypeStruct((2, NPAD, F), jnp.float32),
    mesh=_mesh,
    scratch_types=[
        pltpu.VMEM((CH, CHUNK), jnp.int32),
        pltpu.VMEM((CH, CHUNK), jnp.int32),
        pltpu.VMEM((CHUNK, F), jnp.float32),
        pltpu.VMEM((CHUNK, F), jnp.float32),
        pltpu.VMEM((CHUNK, F), jnp.float32),
        pltpu.VMEM((CHUNK, F), jnp.float32),
        pltpu.SemaphoreType.DMA,
        pltpu.SemaphoreType.DMA,
        pltpu.VMEM_SHARED((NPAD, F), jnp.float32),
    ],
    compiler_params=_sc_params,
)
def _edge_kernel(g_hbm, srcq_hbm, dstq_hbm, zeros_hbm, out_hbm,
                 sidx, didx, rb0, rb1, rb2, rb3, semg, sems, shared):
    c = lax.axis_index("c")
    s = lax.axis_index("s")
    wid = s * 2 + c
    r0 = s * STRIPE
    pltpu.sync_copy(zeros_hbm.at[pl.ds(r0, STRIPE)], shared.at[pl.ds(r0, STRIPE)])
    pltpu.sync_copy(srcq_hbm.at[pl.ds(wid * CH, CH)], sidx)
    pltpu.sync_copy(dstq_hbm.at[pl.ds(wid * CH, CH)], didx)
    plsc.subcore_barrier()

    rows = (rb0, rb1, rb2, rb3)
    for b in range(4):
        pltpu.make_async_copy(g_hbm.at[sidx.at[b]], rows[b], semg).start()

    def body(i, carry):
        for b in range(4):
            m = i * 4 + b
            pltpu.make_async_copy(g_hbm.at[sidx.at[m]], rows[b], semg).wait()
            pltpu.make_async_copy(rows[b], shared.at[didx.at[m]], sems).start(add=True)
        for b in range(4):
            m = i * 4 + b
            pltpu.make_async_copy(rows[b], shared.at[didx.at[m]], sems).wait()
            pltpu.make_async_copy(g_hbm.at[sidx.at[m + 4]], rows[b], semg).start()
        return carry

    lax.fori_loop(0, CH // 4 - 1, body, 0)
    for b in range(4):
        m = CH - 4 + b
        pltpu.make_async_copy(g_hbm.at[sidx.at[m]], rows[b], semg).wait()
        pltpu.make_async_copy(rows[b], shared.at[didx.at[m]], sems).start(add=True)
    for b in range(4):
        m = CH - 4 + b
        pltpu.make_async_copy(rows[b], shared.at[didx.at[m]], sems).wait()
    plsc.subcore_barrier()
    pltpu.sync_copy(shared.at[pl.ds(r0, STRIPE)], out_hbm.at[c, pl.ds(r0, STRIPE)])


def _tc_mm1_body(x_ref, w1_ref, h1_ref):
    h1_ref[:] = jnp.dot(x_ref[:], w1_ref[:], preferred_element_type=jnp.float32)


def _tc_scale_body(h1_ref, degp_ref, g1_ref, dinv_ref):
    degp = degp_ref[:]
    deg = degp[0, :N, 0:1] + degp[1, :N, 0:1] + 1.0
    dinv = lax.rsqrt(deg)
    g1_ref[:] = h1_ref[:] * dinv
    dinv_ref[:] = dinv


def _tc_c_body(t1p_ref, g1_ref, dinv_ref, b1_ref, u2_ref):
    t1p = t1p_ref[:]
    dinv = dinv_ref[:]
    g1 = g1_ref[:]
    h1 = dinv * (t1p[0, :N] + t1p[1, :N] + g1) + b1_ref[:]
    u2_ref[:] = h1 * dinv


def _tc_e_body(t2p_ref, u2_ref, dinv_ref, w2_ref, b2_ref, batch_ref,
               wf1_ref, bf1_ref, wf2_ref, bf2_ref, out_ref):
    t2p = t2p_ref[:]
    dinv = dinv_ref[:]
    m2 = dinv * (t2p[0, :N] + t2p[1, :N] + u2_ref[:])
    h2 = jnp.dot(m2, w2_ref[:], preferred_element_type=jnp.float32) + b2_ref[:]
    ids = lax.broadcasted_iota(jnp.int32, (64, N), 0)
    oh = (ids == batch_ref[:]).astype(jnp.float32)
    sums = jnp.dot(oh, h2, preferred_element_type=jnp.float32)
    cnts = jnp.sum(oh, axis=1, keepdims=True)
    pooled = sums / jnp.maximum(cnts, 1.0)
    z = jnp.maximum(jnp.dot(pooled, wf1_ref[:],
                            preferred_element_type=jnp.float32) + bf1_ref[:], 0.0)
    y = jnp.dot(z, wf2_ref[:], preferred_element_type=jnp.float32) + bf2_ref[:]
    out_ref[:] = jnp.maximum(y, 0.0) + jnp.log1p(jnp.exp(-jnp.abs(y)))


def kernel(x, edge_index, batch, W1, b1, W2, b2, Wf1, bf1, Wf2, bf2):
    src = edge_index[0]
    dst = edge_index[1]
    pad = E_PAD - E
    pr = jnp.arange(pad, dtype=jnp.int32)
    srcq = jnp.concatenate([src, pr % N]).reshape(NW * CH, CHUNK)
    dstq = jnp.concatenate([dst, N + (pr % (NPAD - N))]).reshape(NW * CH, CHUNK)

    zeros16 = jnp.zeros((NPAD, 16), jnp.float32)
    zeros32 = jnp.zeros((NPAD, 32), jnp.float32)
    ones16 = jnp.ones((CHUNK, 16), jnp.float32)

    degp = _deg_kernel(dstq, zeros16, ones16)
    h1 = pl.pallas_call(
        _tc_mm1_body,
        out_shape=jax.ShapeDtypeStruct((N, 32), jnp.float32),
    )(x, W1)

    g1, dinv = pl.pallas_call(
        _tc_scale_body,
        out_shape=(jax.ShapeDtypeStruct((N, 32), jnp.float32),
                   jax.ShapeDtypeStruct((N, 1), jnp.float32)),
    )(h1, degp)

    t1p = _edge_kernel(g1, srcq, dstq, zeros32)

    u2 = pl.pallas_call(
        _tc_c_body,
        out_shape=jax.ShapeDtypeStruct((N, 32), jnp.float32),
    )(t1p, g1, dinv, b1.reshape(1, 32))

    t2p = _edge_kernel(u2, srcq, dstq, zeros32)

    out = pl.pallas_call(
        _tc_e_body,
        out_shape=jax.ShapeDtypeStruct((64, 10), jnp.float32),
    )(t2p, u2, dinv, W2, b2.reshape(1, 64), batch.reshape(1, N),
      Wf1, bf1.reshape(1, 32), Wf2, bf2.reshape(1, 10))

    return out


# speedup vs baseline: 55.6113x; 1.0116x over previous
import functools

import jax
import jax.numpy as jnp
from jax import lax
from jax.experimental import pallas as pl
from jax.experimental.pallas import tpu as pltpu
from jax.experimental.pallas import tpu_sc as plsc

N = 10000
E = 320000
NW = 32
CHUNK = 512
CH = 20
E_PAD = NW * CH * CHUNK
NPAD = 10112
STRIPE = NPAD // 16
LEAD = 8

_mesh = plsc.VectorSubcoreMesh(core_axis_name="c", subcore_axis_name="s")
_sc_params = pltpu.CompilerParams(use_tc_tiling_on_sc=False)


@functools.partial(
    pl.kernel,
    out_type=jax.ShapeDtypeStruct((2, NPAD, 16), jnp.float32),
    mesh=_mesh,
    scratch_types=[
        pltpu.VMEM((CH, CHUNK), jnp.int32),
        pltpu.VMEM((CHUNK, 16), jnp.float32),
        pltpu.SemaphoreType.DMA,
        pltpu.VMEM_SHARED((NPAD, 16), jnp.float32),
    ],
    compiler_params=_sc_params,
)
def _deg_kernel(dstq_hbm, zeros_hbm, ones_hbm, out_hbm, didx, ones_v, sem, shared):
    c = lax.axis_index("c")
    s = lax.axis_index("s")
    wid = s * 2 + c
    r0 = s * STRIPE
    pltpu.sync_copy(zeros_hbm.at[pl.ds(r0, STRIPE)], shared.at[pl.ds(r0, STRIPE)])
    pltpu.sync_copy(dstq_hbm.at[pl.ds(wid * CH, CH)], didx)
    pltpu.sync_copy(ones_hbm, ones_v)
    plsc.subcore_barrier()

    for j in range(LEAD):
        pltpu.make_async_copy(ones_v, shared.at[didx.at[j]], sem).start(add=True)

    def body(j, carry):
        pltpu.make_async_copy(ones_v, shared.at[didx.at[j]], sem).wait()
        pltpu.make_async_copy(ones_v, shared.at[didx.at[j + LEAD]], sem).start(add=True)
        return carry

    lax.fori_loop(0, CH - LEAD, body, 0)
    for j in range(CH - LEAD, CH):
        pltpu.make_async_copy(ones_v, shared.at[didx.at[j]], sem).wait()
    plsc.subcore_barrier()
    pltpu.sync_copy(shared.at[pl.ds(r0, STRIPE)], out_hbm.at[c, pl.ds(r0, STRIPE)])


F = 32

@functools.partial(
    pl.kernel,
    out_type=jax.ShapeDtypeStruct((2, NPAD, F), jnp.float32),
    mesh=_mesh,
    scratch_types=[
        pltpu.VMEM((CH, CHUNK), jnp.int32),
        pltpu.VMEM((CH, CHUNK), jnp.int32),
        pltpu.VMEM((CHUNK, F), jnp.float32),
        pltpu.VMEM((CHUNK, F), jnp.float32),
        pltpu.VMEM((CHUNK, F), jnp.float32),
        pltpu.VMEM((CHUNK, F), jnp.float32),
        pltpu.SemaphoreType.DMA,
        pltpu.SemaphoreType.DMA,
        pltpu.VMEM_SHARED((NPAD, F), jnp.float32),
    ],
    compiler_params=_sc_params,
)
def _edge_kernel(g_hbm, srcq_hbm, dstq_hbm, zeros_hbm, out_hbm,
                 sidx, didx, rb0, rb1, rb2, rb3, semg, sems, shared):
    c = lax.axis_index("c")
    s = lax.axis_index("s")
    wid = s * 2 + c
    r0 = s * STRIPE
    pltpu.sync_copy(zeros_hbm.at[pl.ds(r0, STRIPE)], shared.at[pl.ds(r0, STRIPE)])
    pltpu.sync_copy(srcq_hbm.at[pl.ds(wid * CH, CH)], sidx)
    pltpu.sync_copy(dstq_hbm.at[pl.ds(wid * CH, CH)], didx)
    plsc.subcore_barrier()

    rows = (rb0, rb1, rb2, rb3)
    for b in range(4):
        pltpu.make_async_copy(g_hbm.at[sidx.at[b]], rows[b], semg).start()

    def body(i, carry):
        for b in range(4):
            m = i * 4 + b
            pltpu.make_async_copy(g_hbm.at[sidx.at[m]], rows[b], semg).wait()
            pltpu.make_async_copy(rows[b], shared.at[didx.at[m]], sems).start(add=True)
        for b in range(4):
            m = i * 4 + b
            pltpu.make_async_copy(rows[b], shared.at[didx.at[m]], sems).wait()
            pltpu.make_async_copy(g_hbm.at[sidx.at[m + 4]], rows[b], semg).start()
        return carry

    lax.fori_loop(0, CH // 4 - 1, body, 0)
    for b in range(4):
        m = CH - 4 + b
        pltpu.make_async_copy(g_hbm.at[sidx.at[m]], rows[b], semg).wait()
        pltpu.make_async_copy(rows[b], shared.at[didx.at[m]], sems).start(add=True)
    for b in range(4):
        m = CH - 4 + b
        pltpu.make_async_copy(rows[b], shared.at[didx.at[m]], sems).wait()
    plsc.subcore_barrier()
    pltpu.sync_copy(shared.at[pl.ds(r0, STRIPE)], out_hbm.at[c, pl.ds(r0, STRIPE)])


def _tc_mm1_body(x_ref, w1_ref, h1_ref):
    h1_ref[:] = jnp.dot(x_ref[:], w1_ref[:], preferred_element_type=jnp.float32)


def _tc_scale_body(h1_ref, degp_ref, g1_ref, dinv_ref):
    degp = degp_ref[:]
    deg = degp[0, :N, 0:1] + degp[1, :N, 0:1] + 1.0
    dinv = lax.rsqrt(deg)
    g1_ref[:] = h1_ref[:] * dinv
    dinv_ref[:] = dinv


def _tc_c_body(t1p_ref, g1_ref, dinv_ref, b1_ref, u2_ref):
    t1p = t1p_ref[:]
    dinv = dinv_ref[:]
    g1 = g1_ref[:]
    h1 = dinv * (t1p[0, :N] + t1p[1, :N] + g1) + b1_ref[:]
    u2_ref[:] = h1 * dinv


def _tc_e_body(t2p_ref, u2_ref, dinv_ref, w2_ref, b2_ref, batch_ref,
               wf1_ref, bf1_ref, wf2_ref, bf2_ref, out_ref):
    t2p = t2p_ref[:]
    dinv = dinv_ref[:]
    m2 = dinv * (t2p[0, :N] + t2p[1, :N] + u2_ref[:])
    h2 = jnp.dot(m2, w2_ref[:], preferred_element_type=jnp.float32) + b2_ref[:]
    ids = lax.broadcasted_iota(jnp.int32, (64, N), 0)
    oh = (ids == batch_ref[:]).astype(jnp.float32)
    sums = jnp.dot(oh, h2, preferred_element_type=jnp.float32)
    cnts = jnp.sum(oh, axis=1, keepdims=True)
    pooled = sums / jnp.maximum(cnts, 1.0)
    z = jnp.maximum(jnp.dot(pooled, wf1_ref[:],
                            preferred_element_type=jnp.float32) + bf1_ref[:], 0.0)
    y = jnp.dot(z, wf2_ref[:], preferred_element_type=jnp.float32) + bf2_ref[:]
    out_ref[:] = jnp.maximum(y, 0.0) + jnp.log1p(jnp.exp(-jnp.abs(y)))


def kernel(x, edge_index, batch, W1, b1, W2, b2, Wf1, bf1, Wf2, bf2):
    src = edge_index[0]
    dst = edge_index[1]
    pad = E_PAD - E
    pr = jnp.arange(pad, dtype=jnp.int32)
    srcq = jnp.concatenate([src, pr % N]).reshape(NW * CH, CHUNK)
    dstq = jnp.concatenate([dst, N + (pr % (NPAD - N))]).reshape(NW * CH, CHUNK)

    zeros16 = jnp.zeros((NPAD, 16), jnp.float32)
    zeros32 = jnp.zeros((NPAD, 32), jnp.float32)
    ones16 = jnp.ones((CHUNK, 16), jnp.float32)

    degp = _deg_kernel(dstq, zeros16, ones16)
    h1 = pl.pallas_call(
        _tc_mm1_body,
        out_shape=jax.ShapeDtypeStruct((N, 32), jnp.float32),
    )(x, W1)

    g1, dinv = pl.pallas_call(
        _tc_scale_body,
        out_shape=(jax.ShapeDtypeStruct((N, 32), jnp.float32),
                   jax.ShapeDtypeStruct((N, 1), jnp.float32)),
    )(h1, degp)

    t1p = _edge_kernel(g1, srcq, dstq, zeros32)

    u2 = pl.pallas_call(
        _tc_c_body,
        out_shape=jax.ShapeDtypeStruct((N, 32), jnp.float32),
    )(t1p, g1, dinv, b1.reshape(1, 32))

    t2p = _edge_kernel(u2, srcq, dstq, zeros32)

    out = pl.pallas_call(
        _tc_e_body,
        out_shape=jax.ShapeDtypeStruct((64, 10), jnp.float32),
    )(t2p, u2, dinv, W2, b2.reshape(1, 64), batch.reshape(1, N),
      Wf1, bf1.reshape(1, 32), Wf2, bf2.reshape(1, 10))

    return out
